# baseline (speedup 1.0000x reference)
import functools

import jax
import jax.numpy as jnp
from jax import lax
from jax.experimental import pallas as pl
from jax.experimental.pallas import tpu as pltpu
from jax.experimental.pallas import tpu_sc as plsc

USERS = 100000
ITEMS = 1000000
FACTORS = 128
BATCH = 16384

NUM_CORES = 2
NUM_SUBCORES = 16
NUM_WORKERS = NUM_CORES * NUM_SUBCORES
B_PER_W = BATCH // NUM_WORKERS
CHUNK = 128
N_CHUNKS = B_PER_W // CHUNK

_mesh = plsc.VectorSubcoreMesh(core_axis_name="c", subcore_axis_name="s")


@functools.partial(
    pl.kernel,
    mesh=_mesh,
    out_type=jax.ShapeDtypeStruct((BATCH,), jnp.float32),
    compiler_params=pltpu.CompilerParams(needs_layout_passes=False),
    scratch_types=[
        pltpu.VMEM((B_PER_W,), jnp.int32),
        pltpu.VMEM((B_PER_W,), jnp.int32),
        pltpu.VMEM((CHUNK, FACTORS), jnp.float32),
        pltpu.VMEM((CHUNK, FACTORS), jnp.float32),
        pltpu.VMEM((CHUNK,), jnp.float32),
        pltpu.VMEM((CHUNK,), jnp.float32),
        pltpu.VMEM((CHUNK,), jnp.float32),
        pltpu.SemaphoreType.DMA,
    ],
)
def _mf_sc(user_hbm, item_hbm, uf_hbm, if_hbm, ub_hbm, ib_hbm, out_hbm,
           u_idx, i_idx, uf_buf, if_buf, ub_v, ib_v, out_v, sem):
    wid = lax.axis_index("s") * NUM_CORES + lax.axis_index("c")
    base = wid * B_PER_W

    pltpu.sync_copy(user_hbm.at[pl.ds(base, B_PER_W)], u_idx)
    pltpu.sync_copy(item_hbm.at[pl.ds(base, B_PER_W)], i_idx)

    for c in range(N_CHUNKS):
        iu = u_idx.at[pl.ds(c * CHUNK, CHUNK)]
        ii = i_idx.at[pl.ds(c * CHUNK, CHUNK)]
        cps = [
            pltpu.async_copy(uf_hbm.at[iu], uf_buf, sem),
            pltpu.async_copy(if_hbm.at[ii], if_buf, sem),
            pltpu.async_copy(ub_hbm.at[iu], ub_v, sem),
            pltpu.async_copy(ib_hbm.at[ii], ib_v, sem),
        ]
        for cp in cps:
            cp.wait()

        lane = lax.iota(jnp.int32, 16)

        def gbody(g, carry):
            vec = jnp.zeros((16,), jnp.float32)
            for k in range(16):
                e = g * 16 + k
                acc = uf_buf[e, pl.ds(0, 16)] * if_buf[e, pl.ds(0, 16)]
                for j in range(1, FACTORS // 16):
                    acc = acc + uf_buf[e, pl.ds(j * 16, 16)] * if_buf[e, pl.ds(j * 16, 16)]
                s = jnp.sum(acc)
                vec = jnp.where(lane == k, s, vec)
            gb = g * 16
            out_v[pl.ds(gb, 16)] = vec + ub_v[pl.ds(gb, 16)] + ib_v[pl.ds(gb, 16)]
            return carry

        lax.fori_loop(0, CHUNK // 16, gbody, 0)
        pltpu.sync_copy(out_v, out_hbm.at[pl.ds(base + c * CHUNK, CHUNK)])


def kernel(user, item, user_factor, item_factor, user_biases, item_biases):
    return _mf_sc(
        user.astype(jnp.int32),
        item.astype(jnp.int32),
        user_factor,
        item_factor,
        user_biases.reshape(-1),
        item_biases.reshape(-1),
    )
